# Initial kernel scaffold; baseline (speedup 1.0000x reference)
import jax
import jax.numpy as jnp
from jax.experimental import pallas as pl


def kernel(h, graph_ids, fc_w, fc_b, cls_w, cls_b):
    raise NotImplementedError("write your pallas kernel here")


# speedup vs baseline: 2.7168x; 2.7168x over previous
import functools

import jax
import jax.numpy as jnp
from jax import lax
from jax.experimental import pallas as pl
from jax.experimental.pallas import tpu as pltpu
from jax.experimental.pallas import tpu_sc as plsc

N = 100000
D = 256
G = 1024
NW = 32
ROWS_PER_W = N // NW
BP = 128
NBLK = 25
LOAD_PER_W = NBLK * BP
GPAD = G + 8
CL = 16
STRIPE = G // 16


_SC_MESH = plsc.VectorSubcoreMesh(core_axis_name="c", subcore_axis_name="s")


@functools.partial(
    pl.kernel,
    mesh=_SC_MESH,
    out_type=[
        jax.ShapeDtypeStruct((2 * G, D), jnp.float32),
        jax.ShapeDtypeStruct((2 * G, CL), jnp.float32),
    ],
    scratch_types=[
        pltpu.VMEM((BP,), jnp.int32),
        pltpu.VMEM((BP, D), jnp.float32),
        pltpu.VMEM((BP, CL), jnp.float32),
        pltpu.VMEM_SHARED((GPAD, D), jnp.float32),
        pltpu.VMEM_SHARED((GPAD, CL), jnp.float32),
    ],
    compiler_params=pltpu.CompilerParams(use_tc_tiling_on_sc=False),
)
def _seg_sum_sc(ids_hbm, h_hbm, zsum_hbm, zcnt_hbm, ones_hbm,
                sums_hbm, cnts_hbm,
                idx_v, buf_v, ones_v, acc_s, cnt_s):
    cid = lax.axis_index("c")
    sid = lax.axis_index("s")
    wid = sid * 2 + cid

    pltpu.sync_copy(ones_hbm, ones_v)
    pltpu.sync_copy(zsum_hbm.at[pl.ds(sid * STRIPE, STRIPE)],
                    acc_s.at[pl.ds(sid * STRIPE, STRIPE)])
    pltpu.sync_copy(zcnt_hbm.at[pl.ds(sid * STRIPE, STRIPE)],
                    cnt_s.at[pl.ds(sid * STRIPE, STRIPE)])
    plsc.subcore_barrier()

    row0 = jnp.minimum(wid * ROWS_PER_W // 8 * 8, N - LOAD_PER_W)

    def body(b, carry):
        pltpu.sync_copy(ids_hbm.at[pl.ds(wid * LOAD_PER_W + b * BP, BP)],
                        idx_v)
        pltpu.sync_copy(h_hbm.at[pl.ds(row0 + b * BP, BP)], buf_v)
        pltpu.sync_copy(buf_v, acc_s.at[idx_v], add=True)
        pltpu.sync_copy(ones_v, cnt_s.at[idx_v], add=True)
        return carry

    lax.fori_loop(0, NBLK, body, 0)
    plsc.subcore_barrier()

    pltpu.sync_copy(acc_s.at[pl.ds(sid * STRIPE, STRIPE)],
                    sums_hbm.at[pl.ds(cid * G + sid * STRIPE, STRIPE)])
    pltpu.sync_copy(cnt_s.at[pl.ds(sid * STRIPE, STRIPE)],
                    cnts_hbm.at[pl.ds(cid * G + sid * STRIPE, STRIPE)])


def _mlp_body(sums_ref, cnts_ref, fcw_ref, fcb_ref, clsw_ref, clsb_ref,
              out_ref):
    sums = sums_ref[0] + sums_ref[1]
    cnt = cnts_ref[0] + cnts_ref[1]
    cnt0 = jnp.maximum(cnt[:, 0:1], 1.0)
    gf = sums / cnt0
    hidden = jnp.maximum(jnp.dot(gf, fcw_ref[...]) + fcb_ref[...], 0.0)
    out_ref[...] = jnp.dot(hidden, clsw_ref[...]) + clsb_ref[...]


def _build_ids(graph_ids):
    gid = graph_ids.astype(jnp.int32)
    w = jnp.arange(NW, dtype=jnp.int32)
    start = jnp.minimum(w * ROWS_PER_W // 8 * 8, N - LOAD_PER_W)
    gidx = (start[:, None, None]
            + jnp.arange(NBLK, dtype=jnp.int32)[None, :, None] * BP
            + jnp.arange(BP, dtype=jnp.int32)[None, None, :])
    lo = (w * ROWS_PER_W)[:, None, None]
    hi = lo + ROWS_PER_W
    owned = (gidx >= lo) & (gidx < hi)
    return jnp.where(owned, gid[gidx], G).reshape(-1)


def kernel(h, graph_ids, fc_w, fc_b, cls_w, cls_b):
    ids = _build_ids(graph_ids)
    zsum = jnp.zeros((G, D), jnp.float32)
    zcnt = jnp.zeros((G, CL), jnp.float32)
    ones = jnp.ones((BP, CL), jnp.float32)

    sums2, cnts2 = _seg_sum_sc(ids, h, zsum, zcnt, ones)

    out = pl.pallas_call(
        _mlp_body,
        out_shape=jax.ShapeDtypeStruct((G, 16), jnp.float32),
    )(sums2.reshape(2, G, D), cnts2.reshape(2, G, CL),
      fc_w, fc_b.reshape(1, 512), cls_w, cls_b.reshape(1, 16))
    return out


# speedup vs baseline: 3.1923x; 1.1750x over previous
import functools

import jax
import jax.numpy as jnp
from jax import lax
from jax.experimental import pallas as pl
from jax.experimental.pallas import tpu as pltpu
from jax.experimental.pallas import tpu_sc as plsc

N = 100000
D = 256
G = 1024
NW = 32
ROWS_PER_W = N // NW
BP = 128
NBLK = 25
LOAD_PER_W = NBLK * BP
GPAD = G + 8
CL = 16
STRIPE = G // 16


_SC_MESH = plsc.VectorSubcoreMesh(core_axis_name="c", subcore_axis_name="s")


@functools.partial(
    pl.kernel,
    mesh=_SC_MESH,
    out_type=[
        jax.ShapeDtypeStruct((2 * G, D), jnp.float32),
        jax.ShapeDtypeStruct((2 * G, CL), jnp.float32),
    ],
    scratch_types=[
        pltpu.VMEM((NBLK, BP), jnp.int32),
        pltpu.VMEM((BP, D), jnp.float32),
        pltpu.VMEM((BP, D), jnp.float32),
        pltpu.VMEM((BP, D), jnp.float32),
        pltpu.VMEM((BP, CL), jnp.float32),
        pltpu.VMEM_SHARED((GPAD, D), jnp.float32),
        pltpu.VMEM_SHARED((GPAD, CL), jnp.float32),
        pltpu.SemaphoreType.DMA,
        pltpu.SemaphoreType.DMA,
        pltpu.SemaphoreType.DMA,
        pltpu.SemaphoreType.DMA,
        pltpu.SemaphoreType.DMA,
        pltpu.SemaphoreType.DMA,
    ],
    compiler_params=pltpu.CompilerParams(use_tc_tiling_on_sc=False),
)
def _seg_sum_sc(ids_hbm, h_hbm, zsum_hbm, zcnt_hbm, ones_hbm,
                sums_hbm, cnts_hbm,
                ids_v, buf0, buf1, buf2, ones_v, acc_s, cnt_s,
                ld0, ld1, ld2, st0, st1, st2):
    cid = lax.axis_index("c")
    sid = lax.axis_index("s")
    wid = sid * 2 + cid
    bufs = (buf0, buf1, buf2)
    lds = (ld0, ld1, ld2)
    sts = (st0, st1, st2)

    pltpu.sync_copy(ids_hbm.at[wid], ids_v)
    pltpu.sync_copy(ones_hbm, ones_v)
    pltpu.sync_copy(zsum_hbm.at[pl.ds(sid * STRIPE, STRIPE)],
                    acc_s.at[pl.ds(sid * STRIPE, STRIPE)])
    pltpu.sync_copy(zcnt_hbm.at[pl.ds(sid * STRIPE, STRIPE)],
                    cnt_s.at[pl.ds(sid * STRIPE, STRIPE)])
    plsc.subcore_barrier()

    row0 = jnp.minimum(wid * ROWS_PER_W // 8 * 8, N - LOAD_PER_W)

    def h_src(b):
        return h_hbm.at[pl.ds(row0 + b * BP, BP)]

    def start_scat(b, k):
        pltpu.async_copy(bufs[k], acc_s.at[ids_v.at[b]], sts[k], add=True)
        pltpu.async_copy(ones_v, cnt_s.at[ids_v.at[b]], sts[k], add=True)

    def wait_scat(b, k):
        pltpu.make_async_copy(bufs[k], acc_s.at[ids_v.at[b]], sts[k]).wait()
        pltpu.make_async_copy(ones_v, cnt_s.at[ids_v.at[b]], sts[k]).wait()

    pltpu.async_copy(h_src(0), bufs[0], lds[0])

    def group(g, carry):
        for k in range(3):
            b = g * 3 + k
            kn = (k + 1) % 3
            @pl.when(b >= 2)
            def _():
                wait_scat(b - 2, kn)
            pltpu.async_copy(h_src(b + 1), bufs[kn], lds[kn])
            pltpu.make_async_copy(h_src(b), bufs[k], lds[k]).wait()
            start_scat(b, k)
        return carry

    lax.fori_loop(0, (NBLK - 1) // 3, group, 0)

    bl = NBLK - 1
    kl = bl % 3
    pltpu.make_async_copy(h_src(bl), bufs[kl], lds[kl]).wait()
    start_scat(bl, kl)
    wait_scat(bl - 2, (kl + 1) % 3)
    wait_scat(bl - 1, (kl + 2) % 3)
    wait_scat(bl, kl)
    plsc.subcore_barrier()

    pltpu.sync_copy(acc_s.at[pl.ds(sid * STRIPE, STRIPE)],
                    sums_hbm.at[pl.ds(cid * G + sid * STRIPE, STRIPE)])
    pltpu.sync_copy(cnt_s.at[pl.ds(sid * STRIPE, STRIPE)],
                    cnts_hbm.at[pl.ds(cid * G + sid * STRIPE, STRIPE)])


def _mlp_body(sums_ref, cnts_ref, fcw_ref, fcb_ref, clsw_ref, clsb_ref,
              out_ref):
    sums = sums_ref[0] + sums_ref[1]
    cnt = cnts_ref[0] + cnts_ref[1]
    cnt0 = jnp.maximum(cnt[:, 0:1], 1.0)
    gf = sums / cnt0
    hidden = jnp.maximum(jnp.dot(gf, fcw_ref[...]) + fcb_ref[...], 0.0)
    out_ref[...] = jnp.dot(hidden, clsw_ref[...]) + clsb_ref[...]


def _build_ids(graph_ids):
    gid = graph_ids.astype(jnp.int32)
    w = jnp.arange(NW, dtype=jnp.int32)
    start = jnp.minimum(w * ROWS_PER_W // 8 * 8, N - LOAD_PER_W)
    gidx = (start[:, None, None]
            + jnp.arange(NBLK, dtype=jnp.int32)[None, :, None] * BP
            + jnp.arange(BP, dtype=jnp.int32)[None, None, :])
    lo = (w * ROWS_PER_W)[:, None, None]
    hi = lo + ROWS_PER_W
    owned = (gidx >= lo) & (gidx < hi)
    return jnp.where(owned, gid[gidx], G)


def kernel(h, graph_ids, fc_w, fc_b, cls_w, cls_b):
    ids = _build_ids(graph_ids)
    zsum = jnp.zeros((G, D), jnp.float32)
    zcnt = jnp.zeros((G, CL), jnp.float32)
    ones = jnp.ones((BP, CL), jnp.float32)

    sums2, cnts2 = _seg_sum_sc(ids, h, zsum, zcnt, ones)

    out = pl.pallas_call(
        _mlp_body,
        out_shape=jax.ShapeDtypeStruct((G, 16), jnp.float32),
    )(sums2.reshape(2, G, D), cnts2.reshape(2, G, CL),
      fc_w, fc_b.reshape(1, 512), cls_w, cls_b.reshape(1, 16))
    return out


# speedup vs baseline: 3.4640x; 1.0851x over previous
import functools

import jax
import jax.numpy as jnp
import numpy as np
from jax import lax
from jax.experimental import pallas as pl
from jax.experimental.pallas import tpu as pltpu
from jax.experimental.pallas import tpu_sc as plsc

N = 100000
D = 256
G = 1024
NW = 32
ROWS_PER_W = N // NW
BP = 128
NBLK = 25
LOAD_PER_W = NBLK * BP
GPAD = G + 8
CL = 16
STRIPE = G // 16


_SC_MESH = plsc.VectorSubcoreMesh(core_axis_name="c", subcore_axis_name="s")


@functools.partial(
    pl.kernel,
    mesh=_SC_MESH,
    out_type=[
        jax.ShapeDtypeStruct((2 * G, D), jnp.float32),
        jax.ShapeDtypeStruct((2 * G, CL), jnp.float32),
    ],
    scratch_types=[
        pltpu.VMEM((NBLK, BP), jnp.int32),
        pltpu.VMEM((BP, D), jnp.float32),
        pltpu.VMEM((BP, D), jnp.float32),
        pltpu.VMEM((BP, D), jnp.float32),
        pltpu.VMEM((BP, CL), jnp.float32),
        pltpu.VMEM_SHARED((GPAD, D), jnp.float32),
        pltpu.VMEM_SHARED((GPAD, CL), jnp.float32),
        pltpu.SemaphoreType.DMA,
        pltpu.SemaphoreType.DMA,
        pltpu.SemaphoreType.DMA,
        pltpu.SemaphoreType.DMA,
        pltpu.SemaphoreType.DMA,
        pltpu.SemaphoreType.DMA,
    ],
    compiler_params=pltpu.CompilerParams(use_tc_tiling_on_sc=False),
)
def _seg_sum_sc(ids_hbm, h_hbm, zsum_hbm, zcnt_hbm, ones_hbm,
                sums_hbm, cnts_hbm,
                ids_v, buf0, buf1, buf2, ones_v, acc_s, cnt_s,
                ld0, ld1, ld2, st0, st1, st2):
    cid = lax.axis_index("c")
    sid = lax.axis_index("s")
    wid = sid * 2 + cid
    bufs = (buf0, buf1, buf2)
    lds = (ld0, ld1, ld2)
    sts = (st0, st1, st2)

    pltpu.sync_copy(ids_hbm.at[wid], ids_v)
    pltpu.sync_copy(ones_hbm, ones_v)
    pltpu.sync_copy(zsum_hbm.at[pl.ds(sid * STRIPE, STRIPE)],
                    acc_s.at[pl.ds(sid * STRIPE, STRIPE)])
    pltpu.sync_copy(zcnt_hbm.at[pl.ds(sid * STRIPE, STRIPE)],
                    cnt_s.at[pl.ds(sid * STRIPE, STRIPE)])
    plsc.subcore_barrier()

    row0 = jnp.minimum(wid * ROWS_PER_W // 8 * 8, N - LOAD_PER_W)

    def h_src(b):
        return h_hbm.at[pl.ds(row0 + b * BP, BP)]

    def start_scat(b, k):
        pltpu.async_copy(bufs[k], acc_s.at[ids_v.at[b]], sts[k], add=True)
        pltpu.async_copy(ones_v, cnt_s.at[ids_v.at[b]], sts[k], add=True)

    def wait_scat(b, k):
        pltpu.make_async_copy(bufs[k], acc_s.at[ids_v.at[b]], sts[k]).wait()
        pltpu.make_async_copy(ones_v, cnt_s.at[ids_v.at[b]], sts[k]).wait()

    pltpu.async_copy(h_src(0), bufs[0], lds[0])

    def group(g, carry):
        for k in range(3):
            b = g * 3 + k
            kn = (k + 1) % 3
            @pl.when(b >= 2)
            def _():
                wait_scat(b - 2, kn)
            pltpu.async_copy(h_src(b + 1), bufs[kn], lds[kn])
            pltpu.make_async_copy(h_src(b), bufs[k], lds[k]).wait()
            start_scat(b, k)
        return carry

    lax.fori_loop(0, (NBLK - 1) // 3, group, 0)

    bl = NBLK - 1
    kl = bl % 3
    pltpu.make_async_copy(h_src(bl), bufs[kl], lds[kl]).wait()
    start_scat(bl, kl)
    wait_scat(bl - 2, (kl + 1) % 3)
    wait_scat(bl - 1, (kl + 2) % 3)
    wait_scat(bl, kl)
    plsc.subcore_barrier()

    pltpu.sync_copy(acc_s.at[pl.ds(sid * STRIPE, STRIPE)],
                    sums_hbm.at[pl.ds(cid * G + sid * STRIPE, STRIPE)])
    pltpu.sync_copy(cnt_s.at[pl.ds(sid * STRIPE, STRIPE)],
                    cnts_hbm.at[pl.ds(cid * G + sid * STRIPE, STRIPE)])


def _mlp_body(sums_ref, cnts_ref, fcw_ref, fcb_ref, clsw_ref, clsb_ref,
              out_ref):
    sums = sums_ref[0] + sums_ref[1]
    cnt = cnts_ref[0] + cnts_ref[1]
    cnt0 = jnp.maximum(cnt[:, 0:1], 1.0)
    gf = sums / cnt0
    hidden = jnp.maximum(jnp.dot(gf, fcw_ref[...]) + fcb_ref[...], 0.0)
    out_ref[...] = jnp.dot(hidden, clsw_ref[...]) + clsb_ref[...]


_STARTS = [min(w * ROWS_PER_W // 8 * 8, N - LOAD_PER_W) for w in range(NW)]
_OWNED = np.stack([
    (np.arange(s, s + LOAD_PER_W) >= w * ROWS_PER_W)
    & (np.arange(s, s + LOAD_PER_W) < (w + 1) * ROWS_PER_W)
    for w, s in enumerate(_STARTS)
])


def _build_ids(graph_ids):
    gid = graph_ids.astype(jnp.int32)
    wins = jnp.stack([lax.slice(gid, (s,), (s + LOAD_PER_W,))
                      for s in _STARTS])
    ids = jnp.where(_OWNED, wins, G)
    return ids.reshape(NW, NBLK, BP)


def kernel(h, graph_ids, fc_w, fc_b, cls_w, cls_b):
    ids = _build_ids(graph_ids)
    zsum = jnp.zeros((G, D), jnp.float32)
    zcnt = jnp.zeros((G, CL), jnp.float32)
    ones = jnp.ones((BP, CL), jnp.float32)

    sums2, cnts2 = _seg_sum_sc(ids, h, zsum, zcnt, ones)

    out = pl.pallas_call(
        _mlp_body,
        out_shape=jax.ShapeDtypeStruct((G, 16), jnp.float32),
    )(sums2.reshape(2, G, D), cnts2.reshape(2, G, CL),
      fc_w, fc_b.reshape(1, 512), cls_w, cls_b.reshape(1, 16))
    return out
